# baseline (speedup 1.0000x reference)
import functools

import jax
import jax.numpy as jnp
from jax import lax
from jax.experimental import pallas as pl
from jax.experimental.pallas import tpu as pltpu
from jax.experimental.pallas import tpu_sc as plsc

NC = 2
NS = 16
NW = NC * NS

BATCH = 4096
MAX_LEN = 200
DIM = 64
B = BATCH * MAX_LEN
RPW = BATCH // NW
BPW = B // NW
S = 8
NCK = MAX_LEN // S
PADDING_ID = 1000001


def _body(ids_hbm, pads_hbm, table_hbm, out_hbm,
          idx_v, pads_v, padidx_v, padfill_v, rows_v,
          gsem, osem, psem):
    wid = lax.axis_index("s") * NC + lax.axis_index("c")
    base = wid * BPW
    pltpu.sync_copy(ids_hbm.at[pl.ds(wid * (RPW // 8), RPW // 8)], idx_v)
    pltpu.sync_copy(pads_hbm.at[pl.ds(wid * RPW, RPW)], pads_v)

    pid = jnp.full((16,), PADDING_ID, dtype=jnp.int32)
    for off in range(0, 48, 16):
        padidx_v[pl.ds(off, 16)] = pid
    pltpu.async_copy(
        table_hbm.at[padidx_v.at[pl.ds(0, S)]], padfill_v, psem).wait()

    lanes = lax.iota(jnp.int32, 16)

    def trip_count(r):
        pvec = pads_v[pl.ds(16 * (r // 16), 16)]
        tvec = (MAX_LEN + S - 1 - pvec) // S
        return jnp.max(jnp.where(lanes == (r % 16), tvec, 0))

    def drain_write(vo):
        pltpu.make_async_copy(
            rows_v.at[pl.ds(vo, S)],
            out_hbm.at[wid * RPW, 0], osem).wait()

    def do_row(r):
        t = trip_count(r)
        o = base + r * MAX_LEN
        vo = (r % 2) * MAX_LEN

        rb = r // 8
        sl = r % 8

        def fire(k, carry):
            pltpu.async_copy(
                table_hbm.at[idx_v.at[rb, k * S // 128, sl,
                                      pl.ds((k * S) % 128, S)]],
                rows_v.at[pl.ds(vo + k * S, S)], gsem)
            return carry

        lax.fori_loop(0, t, fire, 0)

        bb = wid * RPW + r

        def wait_and_write(k, carry):
            pltpu.make_async_copy(
                table_hbm.at[idx_v.at[rb, k * S // 128, sl,
                                      pl.ds((k * S) % 128, S)]],
                rows_v.at[pl.ds(vo + k * S, S)], gsem).wait()
            pltpu.async_copy(
                rows_v.at[pl.ds(vo + k * S, S)],
                out_hbm.at[bb, k], osem)
            return carry

        lax.fori_loop(0, t, wait_and_write, 0)

        def fill(k, carry):
            pltpu.async_copy(
                padfill_v, out_hbm.at[bb, k], osem)
            return carry

        lax.fori_loop(t, NCK, fill, 0)

    do_row(0)
    do_row(1)

    def row_loop(r, carry):
        for _ in range(NCK):
            drain_write((r % 2) * MAX_LEN)
        do_row(r)
        return carry

    lax.fori_loop(2, RPW, row_loop, 0)

    for r in (RPW - 2, RPW - 1):
        for _ in range(NCK):
            drain_write((r % 2) * MAX_LEN)


@jax.jit
def _gather(ids_flat, pads, table):
    mesh = plsc.VectorSubcoreMesh(
        core_axis_name="c", subcore_axis_name="s",
        num_cores=NC, num_subcores=NS)
    run = functools.partial(
        pl.kernel, mesh=mesh,
        compiler_params=pltpu.CompilerParams(use_tc_tiling_on_sc=False, needs_layout_passes=False),
        out_type=jax.ShapeDtypeStruct((BATCH, NCK, S, 128), jnp.float32),
        scratch_types=(
            [pltpu.VMEM((RPW // 8, 2, 8, 128), jnp.int32),
             pltpu.VMEM((RPW,), jnp.int32),
             pltpu.VMEM((48,), jnp.int32),
             pltpu.VMEM((S, 128), jnp.float32),
             pltpu.VMEM((2 * MAX_LEN, 128), jnp.float32),
             pltpu.SemaphoreType.DMA,
             pltpu.SemaphoreType.DMA,
             pltpu.SemaphoreType.DMA]
        ))(_body)
    return run(ids_flat, pads, table)


def kernel(torch_ids, pads, table):
    ids_t = jnp.pad(torch_ids, ((0, 0), (0, 256 - MAX_LEN)))
    ids_t = ids_t.reshape(BATCH // 8, 8, 2, 128).transpose(0, 2, 1, 3)
    table_p = jnp.pad(table, ((0, 6), (0, 128 - DIM)))
    out = _gather(ids_t, pads, table_p)
    out = out.reshape(BATCH, MAX_LEN, 128)[:, :, :DIM]
    return out, pads


# speedup vs baseline: 1.3644x; 1.3644x over previous
import functools

import jax
import jax.numpy as jnp
from jax import lax
from jax.experimental import pallas as pl
from jax.experimental.pallas import tpu as pltpu
from jax.experimental.pallas import tpu_sc as plsc

NC = 2
NS = 16
NW = NC * NS

BATCH = 4096
MAX_LEN = 200
DIM = 64
B = BATCH * MAX_LEN
RPW = BATCH // NW
BPW = B // NW
S = 8
NCK = MAX_LEN // S
PADDING_ID = 1000001


def _body(ids_hbm, pads_hbm, table_hbm, out_hbm,
          idx_v, pads_v, padidx_v, padfill_v, rows_v,
          gsem, osem, psem):
    wid = lax.axis_index("s") * NC + lax.axis_index("c")
    base = wid * BPW
    pltpu.sync_copy(ids_hbm.at[pl.ds(wid * (RPW // 8), RPW // 8)], idx_v)
    pltpu.sync_copy(pads_hbm.at[pl.ds(wid * RPW, RPW)], pads_v)

    pid = jnp.full((16,), PADDING_ID, dtype=jnp.int32)
    for off in range(0, 48, 16):
        padidx_v[pl.ds(off, 16)] = pid
    pltpu.async_copy(
        table_hbm.at[padidx_v.at[pl.ds(0, S)]], padfill_v, psem).wait()

    lanes = lax.iota(jnp.int32, 16)

    def trip_count(r):
        pvec = pads_v[pl.ds(16 * (r // 16), 16)]
        tvec = (MAX_LEN + S - 1 - pvec) // S
        return jnp.max(jnp.where(lanes == (r % 16), tvec, 0))

    def drain_write(vo):
        pltpu.make_async_copy(
            rows_v.at[pl.ds(vo, S)],
            out_hbm.at[wid * RPW, 0, :, pl.ds(0, DIM)], osem).wait()

    def fire_row(r):
        t = trip_count(r)
        vo = (r % 2) * MAX_LEN
        rb = r // 8
        sl = r % 8

        def fire(k, carry):
            pltpu.async_copy(
                table_hbm.at[idx_v.at[rb, k * S // 128, sl,
                                      pl.ds((k * S) % 128, S)]],
                rows_v.at[pl.ds(vo + k * S, S)], gsem)
            return carry

        lax.fori_loop(0, t, fire, 0)

    def complete_row(r):
        t = trip_count(r)
        vo = (r % 2) * MAX_LEN
        rb = r // 8
        sl = r % 8
        bb = wid * RPW + r

        def wait_and_write(k, carry):
            pltpu.make_async_copy(
                table_hbm.at[idx_v.at[rb, k * S // 128, sl,
                                      pl.ds((k * S) % 128, S)]],
                rows_v.at[pl.ds(vo + k * S, S)], gsem).wait()
            pltpu.async_copy(
                rows_v.at[pl.ds(vo + k * S, S)],
                out_hbm.at[bb, k, :, pl.ds(0, DIM)], osem)
            return carry

        lax.fori_loop(0, t, wait_and_write, 0)

        def fill(k, carry):
            pltpu.async_copy(
                padfill_v, out_hbm.at[bb, k, :, pl.ds(0, DIM)], osem)
            return carry

        lax.fori_loop(t, NCK, fill, 0)

    fire_row(0)
    fire_row(1)
    complete_row(0)

    def row_loop(r, carry):
        for _ in range(NCK):
            drain_write((r % 2) * MAX_LEN)
        fire_row(r)
        complete_row(r - 1)
        return carry

    lax.fori_loop(2, RPW, row_loop, 0)

    complete_row(RPW - 1)
    for r in (RPW - 2, RPW - 1):
        for _ in range(NCK):
            drain_write((r % 2) * MAX_LEN)


@jax.jit
def _gather(ids_flat, pads, table):
    mesh = plsc.VectorSubcoreMesh(
        core_axis_name="c", subcore_axis_name="s",
        num_cores=NC, num_subcores=NS)
    run = functools.partial(
        pl.kernel, mesh=mesh,
        compiler_params=pltpu.CompilerParams(use_tc_tiling_on_sc=False, needs_layout_passes=False),
        out_type=jax.ShapeDtypeStruct((BATCH, NCK, S, 128), jnp.float32),
        scratch_types=(
            [pltpu.VMEM((RPW // 8, 2, 8, 128), jnp.int32),
             pltpu.VMEM((RPW,), jnp.int32),
             pltpu.VMEM((48,), jnp.int32),
             pltpu.VMEM((S, DIM), jnp.float32),
             pltpu.VMEM((2 * MAX_LEN, DIM), jnp.float32),
             pltpu.SemaphoreType.DMA,
             pltpu.SemaphoreType.DMA,
             pltpu.SemaphoreType.DMA]
        ))(_body)
    return run(ids_flat, pads, table)


def kernel(torch_ids, pads, table):
    ids_t = jnp.pad(torch_ids, ((0, 0), (0, 256 - MAX_LEN)))
    ids_t = ids_t.reshape(BATCH // 8, 8, 2, 128).transpose(0, 2, 1, 3)
    out = _gather(ids_t, pads, table)
    out = out.reshape(BATCH, MAX_LEN, 128)[:, :, :DIM]
    return out, pads
